# Initial kernel scaffold; baseline (speedup 1.0000x reference)
import jax
import jax.numpy as jnp
from jax.experimental import pallas as pl


def kernel(hidden_states, pe_k_weight):
    raise NotImplementedError("write your pallas kernel here")


# speedup vs baseline: 7.6259x; 7.6259x over previous
import functools

import jax
import jax.numpy as jnp
from jax.experimental import pallas as pl
from jax.experimental.pallas import tpu as pltpu

_DIM = 64
_MAX_LENGTH = 160
_TBL = 2 * _MAX_LENGTH
_BI = 8


def _body(tbl_hbm, out_hbm, tbl_v, c_v, sem, row_sems, *, seq):
    pid = pl.program_id(0)

    @pl.when(pid == 0)
    def _build_strip():
        cp = pltpu.make_async_copy(tbl_hbm, tbl_v, sem)
        cp.start()
        cp.wait()
        n_hi = seq - _MAX_LENGTH
        c_v[pl.ds(0, n_hi), :] = jnp.broadcast_to(
            tbl_v[pl.ds(_TBL - 1, 1), :], (n_hi, _DIM))
        n_lo = 2 * seq - (n_hi + _TBL)
        c_v[pl.ds(n_hi + _TBL, n_lo), :] = jnp.broadcast_to(
            tbl_v[pl.ds(0, 1), :], (n_lo, _DIM))

        def _rev(k, carry):
            c_v[pl.ds(n_hi + k, 1), :] = tbl_v[pl.ds(_TBL - 1 - k, 1), :]
            return carry

        jax.lax.fori_loop(0, _TBL, _rev, 0)

    base = pid * _BI
    for r in range(_BI):
        i = base + r
        pltpu.make_async_copy(
            c_v.at[pl.ds(seq - 1 - i, seq)], out_hbm.at[i], row_sems.at[r]
        ).start()
    for r in range(_BI):
        i = base + r
        pltpu.make_async_copy(
            c_v.at[pl.ds(seq - 1 - i, seq)], out_hbm.at[i], row_sems.at[r]
        ).wait()


def kernel(hidden_states, pe_k_weight):
    seq = hidden_states.shape[1]
    return pl.pallas_call(
        functools.partial(_body, seq=seq),
        grid=(seq // _BI,),
        in_specs=[pl.BlockSpec(memory_space=pl.ANY)],
        out_specs=pl.BlockSpec(memory_space=pl.ANY),
        out_shape=jax.ShapeDtypeStruct((seq, seq, _DIM), jnp.float32),
        scratch_shapes=[
            pltpu.VMEM((_TBL, _DIM), jnp.float32),
            pltpu.VMEM((2 * seq, _DIM), jnp.float32),
            pltpu.SemaphoreType.DMA,
            pltpu.SemaphoreType.DMA((_BI,)),
        ],
        compiler_params=pltpu.CompilerParams(
            dimension_semantics=("arbitrary",)),
    )(pe_k_weight)


# speedup vs baseline: 8.3002x; 1.0884x over previous
import functools

import jax
import jax.numpy as jnp
from jax.experimental import pallas as pl
from jax.experimental.pallas import tpu as pltpu

_DIM = 64
_MAX_LENGTH = 160
_TBL = 2 * _MAX_LENGTH
_BI = 8
_LAG = 4


def _body(tbl_hbm, out_hbm, tbl_v, c_v, strip_sem, sem, *, seq):
    pid = pl.program_id(0)

    @pl.when(pid == 0)
    def _build_strip():
        cp = pltpu.make_async_copy(tbl_hbm, tbl_v, strip_sem)
        cp.start()
        cp.wait()
        n_hi = seq - _MAX_LENGTH
        c_v[pl.ds(0, n_hi), :] = jnp.broadcast_to(
            tbl_v[pl.ds(_TBL - 1, 1), :], (n_hi, _DIM))
        n_lo = 2 * seq - (n_hi + _TBL)
        c_v[pl.ds(n_hi + _TBL, n_lo), :] = jnp.broadcast_to(
            tbl_v[pl.ds(0, 1), :], (n_lo, _DIM))

        def _rev(k, carry):
            c_v[pl.ds(n_hi + k, 1), :] = tbl_v[pl.ds(_TBL - 1 - k, 1), :]
            return carry

        jax.lax.fori_loop(0, _TBL, _rev, 0)

    base = pid * _BI
    nsteps = pl.num_programs(0)

    def _cp(i):
        return pltpu.make_async_copy(
            c_v.at[pl.ds(seq - 1 - i, seq)], out_hbm.at[i], sem)

    for r in range(_BI):
        _cp(base + r).start()

    @pl.when(pid >= _LAG)
    def _reclaim():
        for r in range(_BI):
            _cp((pid - _LAG) * _BI + r).wait()

    @pl.when(pid == nsteps - 1)
    def _drain():
        for k in range(_LAG):
            for r in range(_BI):
                _cp((nsteps - _LAG + k) * _BI + r).wait()


def kernel(hidden_states, pe_k_weight):
    seq = hidden_states.shape[1]
    return pl.pallas_call(
        functools.partial(_body, seq=seq),
        grid=(seq // _BI,),
        in_specs=[pl.BlockSpec(memory_space=pl.ANY)],
        out_specs=pl.BlockSpec(memory_space=pl.ANY),
        out_shape=jax.ShapeDtypeStruct((seq, seq, _DIM), jnp.float32),
        scratch_shapes=[
            pltpu.VMEM((_TBL, _DIM), jnp.float32),
            pltpu.VMEM((2 * seq, _DIM), jnp.float32),
            pltpu.SemaphoreType.DMA,
            pltpu.SemaphoreType.DMA,
        ],
        compiler_params=pltpu.CompilerParams(
            dimension_semantics=("arbitrary",)),
    )(pe_k_weight)


# speedup vs baseline: 8.4638x; 1.0197x over previous
import functools

import jax
import jax.numpy as jnp
from jax.experimental import pallas as pl
from jax.experimental.pallas import tpu as pltpu

_DIM = 64
_MAX_LENGTH = 160
_TBL = 2 * _MAX_LENGTH
_BI = 8
_LAG = 4


def _body(tbl_hbm, out_hbm, tbl_v, c0_v, c1_v, strip_sem, sem, *, seq):
    pid = pl.program_id(0)
    nsteps = pl.num_programs(0)

    @pl.when(pid == 0)
    def _build():
        cp = pltpu.make_async_copy(tbl_hbm, tbl_v, strip_sem)
        cp.start()
        cp.wait()
        rh = (seq - _MAX_LENGTH) // 2
        nrev = _TBL // 2
        rlo = seq - rh - nrev
        t319 = tbl_v[pl.ds(_TBL - 1, 1), :]
        t0 = tbl_v[pl.ds(0, 1), :]
        for cv in (c0_v, c1_v):
            for lane in (0, _DIM):
                cv[pl.ds(0, rh), pl.ds(lane, _DIM)] = jnp.broadcast_to(
                    t319, (rh, _DIM))
                cv[pl.ds(rh + nrev, rlo), pl.ds(lane, _DIM)] = jnp.broadcast_to(
                    t0, (rlo, _DIM))

        two_rh = 2 * rh

        def _rev(t, carry):
            a = (_TBL - 1) + two_rh - 2 * t
            c0_v[pl.ds(t, 1), pl.ds(0, _DIM)] = tbl_v[pl.ds(a, 1), :]
            c0_v[pl.ds(t, 1), pl.ds(_DIM, _DIM)] = tbl_v[pl.ds(a - 1, 1), :]
            c1_v[pl.ds(t, 1), pl.ds(0, _DIM)] = tbl_v[pl.ds(a - 1, 1), :]
            c1_v[pl.ds(t, 1), pl.ds(_DIM, _DIM)] = tbl_v[
                pl.ds(jnp.maximum(a - 2, 0), 1), :]
            return carry

        jax.lax.fori_loop(rh, rh + nrev, _rev, 0)

    base = pid * _BI
    hseq = seq // 2

    def _cp(i, r):
        m_is_even = (r % 2) == 1
        if m_is_even:
            src = c0_v.at[pl.ds((seq - 1 - i) // 2, hseq), :]
        else:
            src = c1_v.at[pl.ds((seq - 2 - i) // 2, hseq), :]
        return pltpu.make_async_copy(src, out_hbm.at[i], sem)

    for r in range(_BI):
        _cp(base + r, r).start()

    @pl.when(pid >= _LAG)
    def _reclaim():
        for r in range(_BI):
            _cp((pid - _LAG) * _BI + r, r).wait()

    @pl.when(pid == nsteps - 1)
    def _drain():
        for k in range(_LAG):
            for r in range(_BI):
                _cp((nsteps - _LAG + k) * _BI + r, r).wait()


def kernel(hidden_states, pe_k_weight):
    seq = hidden_states.shape[1]
    out = pl.pallas_call(
        functools.partial(_body, seq=seq),
        grid=(seq // _BI,),
        in_specs=[pl.BlockSpec(memory_space=pl.ANY)],
        out_specs=pl.BlockSpec(memory_space=pl.ANY),
        out_shape=jax.ShapeDtypeStruct((seq, seq // 2, 2 * _DIM), jnp.float32),
        scratch_shapes=[
            pltpu.VMEM((_TBL, _DIM), jnp.float32),
            pltpu.VMEM((seq, 2 * _DIM), jnp.float32),
            pltpu.VMEM((seq, 2 * _DIM), jnp.float32),
            pltpu.SemaphoreType.DMA,
            pltpu.SemaphoreType.DMA,
        ],
        compiler_params=pltpu.CompilerParams(
            dimension_semantics=("arbitrary",)),
    )(pe_k_weight)
    return out.reshape(seq, seq, _DIM)
